# Initial kernel scaffold; baseline (speedup 1.0000x reference)
import functools

import jax
import jax.numpy as jnp
from jax import lax
from jax.experimental import pallas as pl
from jax.experimental.pallas import tpu as pltpu
from jax.experimental.pallas import tpu_sc as plsc

N = 10000
E = 320000
D = 128
L = 2

NC = 2
NS = 16
NW = NC * NS
EPW = E // NW
CHUNK = 128
NFULL = EPW // CHUNK
TAIL = EPW - NFULL * CHUNK


def _zero_fill(ref, nwords):
    z = jnp.zeros((16,), jnp.float32)

    @pl.loop(0, nwords, step=16)
    def _(i):
        ref[pl.ds(i, 16)] = z


def _sc_degree(adj_col, adj_data):
    mesh = plsc.VectorSubcoreMesh(core_axis_name="c", subcore_axis_name="s")

    @functools.partial(
        pl.kernel,
        out_type=jax.ShapeDtypeStruct((NC, N), jnp.float32),
        mesh=mesh,
        scratch_types=[
            pltpu.VMEM_SHARED((N,), jnp.float32),
            pltpu.VMEM((CHUNK,), jnp.int32),
            pltpu.VMEM((CHUNK,), jnp.float32),
            pltpu.VMEM((TAIL,), jnp.int32),
            pltpu.VMEM((TAIL,), jnp.float32),
            pltpu.VMEM((N // NS,), jnp.float32),
            pltpu.SemaphoreType.DMA,
        ],
    )
    def k(col_hbm, data_hbm, out_hbm, acc, col_v, dat_v, colt_v, datt_v,
          zbuf, sem):
        c = lax.axis_index("c")
        s = lax.axis_index("s")
        wid = s * NC + c
        base = wid * EPW

        nps = N // NS
        _zero_fill(zbuf, nps)
        pltpu.sync_copy(zbuf, acc.at[pl.ds(s * nps, nps)])
        plsc.subcore_barrier()

        @pl.loop(0, NFULL)
        def _(i):
            off = base + i * CHUNK
            pltpu.sync_copy(col_hbm.at[pl.ds(off, CHUNK)], col_v)
            pltpu.sync_copy(data_hbm.at[pl.ds(off, CHUNK)], dat_v)
            pltpu.async_copy(dat_v, acc.at[col_v], sem, add=True).wait()

        off = base + NFULL * CHUNK
        pltpu.sync_copy(col_hbm.at[pl.ds(off, TAIL)], colt_v)
        pltpu.sync_copy(data_hbm.at[pl.ds(off, TAIL)], datt_v)
        pltpu.async_copy(datt_v, acc.at[colt_v], sem, add=True).wait()

        plsc.subcore_barrier()
        pltpu.sync_copy(acc.at[pl.ds(s * nps, nps)],
                        out_hbm.at[c, pl.ds(s * nps, nps)])

    return k(adj_col, adj_data)


def _sc_spmm(adj_row, adj_col, adj_data, deg, h):
    mesh = plsc.VectorSubcoreMesh(core_axis_name="c", subcore_axis_name="s")

    @functools.partial(
        pl.kernel,
        out_type=jax.ShapeDtypeStruct((NC, N, D), jnp.float32),
        mesh=mesh,
        scratch_types=[
            pltpu.VMEM_SHARED((N, D), jnp.float32),
            pltpu.VMEM((CHUNK,), jnp.int32),
            pltpu.VMEM((CHUNK,), jnp.int32),
            pltpu.VMEM((CHUNK,), jnp.float32),
            pltpu.VMEM((CHUNK,), jnp.float32),
            pltpu.VMEM((CHUNK,), jnp.float32),
            pltpu.VMEM((CHUNK, D), jnp.float32),
            pltpu.VMEM((TAIL,), jnp.int32),
            pltpu.VMEM((TAIL,), jnp.int32),
            pltpu.VMEM((TAIL,), jnp.float32),
            pltpu.VMEM((TAIL,), jnp.float32),
            pltpu.VMEM((TAIL,), jnp.float32),
            pltpu.VMEM((TAIL, D), jnp.float32),
            pltpu.VMEM((N // NS // 5, D), jnp.float32),
            pltpu.SemaphoreType.DMA,
            pltpu.SemaphoreType.DMA,
        ],
    )
    def k(row_hbm, col_hbm, data_hbm, deg_hbm, h_hbm, out_hbm, acc,
          row_v, col_v, dat_v, degg_v, scl_v, rows_v,
          rowt_v, colt_v, datt_v, degt_v, sclt_v, rowst_v,
          zbuf, sem_g, sem_s):
        c = lax.axis_index("c")
        s = lax.axis_index("s")
        wid = s * NC + c
        base = wid * EPW

        nps = N // NS
        zrows = nps // 5
        _zero_fill(zbuf.reshape(zrows * D), zrows * D)

        @pl.loop(0, 5)
        def _(i):
            pltpu.sync_copy(zbuf, acc.at[pl.ds(s * nps + i * zrows, zrows)])

        plsc.subcore_barrier()

        def process_chunk(off, n, rv, cv, dv, gv, sv, rsv):
            pltpu.sync_copy(row_hbm.at[pl.ds(off, n)], rv)
            pltpu.sync_copy(col_hbm.at[pl.ds(off, n)], cv)
            pltpu.sync_copy(data_hbm.at[pl.ds(off, n)], dv)
            cp_rows = pltpu.async_copy(h_hbm.at[cv], rsv, sem_g)
            cp_deg = pltpu.async_copy(deg_hbm.at[cv], gv, sem_g)
            cp_deg.wait()

            @pl.loop(0, n, step=16)
            def _(j):
                sv[pl.ds(j, 16)] = dv[pl.ds(j, 16)] / gv[pl.ds(j, 16)]

            cp_rows.wait()

            @pl.loop(0, n)
            def _(e):
                vv = plsc.load_gather(sv, [jnp.broadcast_to(e, (16,))])
                for d in range(D // 16):
                    rsv[e, pl.ds(d * 16, 16)] = rsv[e, pl.ds(d * 16, 16)] * vv

            pltpu.async_copy(rsv, acc.at[rv], sem_s, add=True).wait()

        @pl.loop(0, NFULL)
        def _(i):
            process_chunk(base + i * CHUNK, CHUNK,
                          row_v, col_v, dat_v, degg_v, scl_v, rows_v)

        process_chunk(base + NFULL * CHUNK, TAIL,
                      rowt_v, colt_v, datt_v, degt_v, sclt_v, rowst_v)

        plsc.subcore_barrier()

        @pl.loop(0, 5)
        def _(i):
            r0 = s * nps + i * zrows
            pltpu.sync_copy(acc.at[pl.ds(r0, zrows)],
                            out_hbm.at[c, pl.ds(r0, zrows)])

    return k(adj_row, adj_col, adj_data, deg, h)


def _tc1_body(dp_ref, emb_ref, w1t_ref, wa_ref, ba_ref, deg_ref, h_ref):
    deg_ref[...] = dp_ref[0] + dp_ref[1]
    x = emb_ref[...]
    scores = jnp.dot(x, wa_ref[...], preferred_element_type=jnp.float32)
    scores = scores + ba_ref[0, 0]
    m = jnp.max(scores)
    ex = jnp.exp(scores - m)
    attn = ex / jnp.sum(ex)
    xw = jnp.dot(x, w1t_ref[...], preferred_element_type=jnp.float32)
    h_ref[...] = xw * attn


def _tc1(deg_partials, emb, w1t, wa, ba):
    return pl.pallas_call(
        _tc1_body,
        out_shape=(jax.ShapeDtypeStruct((N,), jnp.float32),
                   jax.ShapeDtypeStruct((N, D), jnp.float32)),
    )(deg_partials, emb, w1t, wa, ba)


def _norm_rows(x):
    nrm = jnp.sqrt(jnp.sum(x * x, axis=-1, keepdims=True))
    return x / jnp.maximum(nrm, 1e-12)


def _tc2_body(p_ref, emb_ref, w2t_ref, wa_ref, ba_ref, f1_ref, h_ref):
    x = _norm_rows(p_ref[0] + p_ref[1])
    f1_ref[...] = emb_ref[...] + x
    scores = jnp.dot(x, wa_ref[...], preferred_element_type=jnp.float32)
    scores = scores + ba_ref[0, 0]
    m = jnp.max(scores)
    ex = jnp.exp(scores - m)
    attn = ex / jnp.sum(ex)
    xw = jnp.dot(x, w2t_ref[...], preferred_element_type=jnp.float32)
    h_ref[...] = xw * attn


def _tc2(partials, emb, w2t, wa, ba):
    return pl.pallas_call(
        _tc2_body,
        out_shape=(jax.ShapeDtypeStruct((N, D), jnp.float32),
                   jax.ShapeDtypeStruct((N, D), jnp.float32)),
    )(partials, emb, w2t, wa, ba)


def _tc3_body(p_ref, f1_ref, out_ref):
    x = _norm_rows(p_ref[0] + p_ref[1])
    out_ref[...] = (f1_ref[...] + x) * (1.0 / (L + 1))


def _tc3(partials, f1):
    return pl.pallas_call(
        _tc3_body,
        out_shape=jax.ShapeDtypeStruct((N, D), jnp.float32),
    )(partials, f1)


def kernel(adj_row, adj_col, adj_data, embedding, W_item, W_att, b_att):
    adj_row = adj_row.astype(jnp.int32)
    adj_col = adj_col.astype(jnp.int32)
    ba = b_att.reshape(1, 1)

    deg_partials = _sc_degree(adj_col, adj_data)
    deg, h1 = _tc1(deg_partials, embedding, W_item[0].T, W_att, ba)
    p1 = _sc_spmm(adj_row, adj_col, adj_data, deg, h1)
    f1, h2 = _tc2(p1, embedding, W_item[1].T, W_att, ba)
    p2 = _sc_spmm(adj_row, adj_col, adj_data, deg, h2)
    return _tc3(p2, f1)


# speedup vs baseline: 6.4583x; 6.4583x over previous
import functools

import jax
import jax.numpy as jnp
from jax import lax
from jax.experimental import pallas as pl
from jax.experimental.pallas import tpu as pltpu
from jax.experimental.pallas import tpu_sc as plsc

N = 10000
E = 320000
D = 128
L = 2

NC = 2
NS = 16
NW = NC * NS
EPW = E // NW
CHUNK = 128
NFULL = EPW // CHUNK
TAIL = EPW - NFULL * CHUNK


def _zero_fill(ref, nwords):
    z = jnp.zeros((16,), jnp.float32)

    @pl.loop(0, nwords, step=16)
    def _(i):
        ref[pl.ds(i, 16)] = z


def _sc_degree(adj_col, adj_data):
    mesh = plsc.VectorSubcoreMesh(core_axis_name="c", subcore_axis_name="s")

    @functools.partial(
        pl.kernel,
        out_type=jax.ShapeDtypeStruct((NC * N,), jnp.float32),
        mesh=mesh,
        scratch_types=[
            pltpu.VMEM_SHARED((N,), jnp.float32),
            pltpu.VMEM((CHUNK,), jnp.int32),
            pltpu.VMEM((CHUNK,), jnp.float32),
            pltpu.VMEM((TAIL,), jnp.int32),
            pltpu.VMEM((TAIL,), jnp.float32),
            pltpu.VMEM((640,), jnp.float32),
            pltpu.SemaphoreType.DMA,
        ],
    )
    def k(col_hbm, data_hbm, out_hbm, acc, col_v, dat_v, colt_v, datt_v,
          zbuf, sem):
        c = lax.axis_index("c")
        s = lax.axis_index("s")
        wid = s * NC + c
        base = wid * EPW

        _zero_fill(zbuf, 640)
        pltpu.sync_copy(zbuf.at[pl.ds(0, 624)], acc.at[pl.ds(s * 624, 624)])

        @pl.when(s == NS - 1)
        def _():
            pltpu.sync_copy(zbuf.at[pl.ds(0, 16)], acc.at[pl.ds(9984, 16)])

        plsc.subcore_barrier()

        @pl.loop(0, NFULL)
        def _(i):
            off = base + i * CHUNK
            pltpu.sync_copy(col_hbm.at[pl.ds(off, CHUNK)], col_v)
            pltpu.sync_copy(data_hbm.at[pl.ds(off, CHUNK)], dat_v)
            pltpu.async_copy(dat_v, acc.at[col_v], sem, add=True).wait()

        off = base + NFULL * CHUNK
        pltpu.sync_copy(col_hbm.at[pl.ds(off, TAIL)], colt_v)
        pltpu.sync_copy(data_hbm.at[pl.ds(off, TAIL)], datt_v)
        pltpu.async_copy(datt_v, acc.at[colt_v], sem, add=True).wait()

        plsc.subcore_barrier()
        pltpu.sync_copy(acc.at[pl.ds(s * 624, 624)], zbuf.at[pl.ds(0, 624)])
        pltpu.sync_copy(zbuf.at[pl.ds(0, 624)],
                        out_hbm.at[pl.ds(c * N + s * 624, 624)])

        @pl.when(s == NS - 1)
        def _():
            pltpu.sync_copy(acc.at[pl.ds(9984, 16)], zbuf.at[pl.ds(0, 16)])
            pltpu.sync_copy(zbuf.at[pl.ds(0, 16)],
                            out_hbm.at[pl.ds(c * N + 9984, 16)])

    return k(adj_col, adj_data)


def _sc_spmm(adj_row, adj_col, adj_data, deg, h):
    mesh = plsc.VectorSubcoreMesh(core_axis_name="c", subcore_axis_name="s")

    @functools.partial(
        pl.kernel,
        out_type=jax.ShapeDtypeStruct((NC, N, D), jnp.float32),
        mesh=mesh,
        scratch_types=[
            pltpu.VMEM_SHARED((N, D), jnp.float32),
            pltpu.VMEM((CHUNK,), jnp.int32),
            pltpu.VMEM((CHUNK,), jnp.int32),
            pltpu.VMEM((CHUNK,), jnp.float32),
            pltpu.VMEM((CHUNK,), jnp.float32),
            pltpu.VMEM((CHUNK,), jnp.float32),
            pltpu.VMEM((CHUNK, D), jnp.float32),
            pltpu.VMEM((TAIL,), jnp.int32),
            pltpu.VMEM((TAIL,), jnp.int32),
            pltpu.VMEM((TAIL,), jnp.float32),
            pltpu.VMEM((TAIL,), jnp.float32),
            pltpu.VMEM((TAIL,), jnp.float32),
            pltpu.VMEM((TAIL, D), jnp.float32),
            pltpu.VMEM((208, D), jnp.float32),
            pltpu.SemaphoreType.DMA,
            pltpu.SemaphoreType.DMA,
        ],
    )
    def k(row_hbm, col_hbm, data_hbm, deg_hbm, h_hbm, out_hbm, acc,
          row_v, col_v, dat_v, degg_v, scl_v, rows_v,
          rowt_v, colt_v, datt_v, degt_v, sclt_v, rowst_v,
          zbuf, sem_g, sem_s):
        c = lax.axis_index("c")
        s = lax.axis_index("s")
        wid = s * NC + c
        base = wid * EPW

        z = jnp.zeros((16,), jnp.float32)

        @pl.loop(0, 208)
        def _(i):
            for d in range(D // 16):
                zbuf[i, pl.ds(d * 16, 16)] = z

        @pl.loop(0, 3)
        def _(i):
            pltpu.sync_copy(zbuf, acc.at[pl.ds(s * 624 + i * 208, 208)])

        @pl.when(s == NS - 1)
        def _():
            pltpu.sync_copy(zbuf.at[pl.ds(0, 16)], acc.at[pl.ds(9984, 16)])

        plsc.subcore_barrier()

        def process_chunk(off, n, rv, cv, dv, gv, sv, rsv):
            pltpu.sync_copy(row_hbm.at[pl.ds(off, n)], rv)
            pltpu.sync_copy(col_hbm.at[pl.ds(off, n)], cv)
            pltpu.sync_copy(data_hbm.at[pl.ds(off, n)], dv)
            cp_rows = pltpu.async_copy(h_hbm.at[cv], rsv, sem_g)
            cp_deg = pltpu.async_copy(deg_hbm.at[cv], gv, sem_g)
            cp_deg.wait()

            @pl.loop(0, n, step=16)
            def _(j):
                sv[pl.ds(j, 16)] = dv[pl.ds(j, 16)] / gv[pl.ds(j, 16)]

            cp_rows.wait()

            @pl.loop(0, n // 16)
            def _(jj):
                sj = sv[pl.ds(jj * 16, 16)]
                for t in range(16):
                    e = jj * 16 + t
                    vv = lax.gather(
                        sj, jnp.full((16, 1), t, jnp.int32),
                        dimension_numbers=lax.GatherDimensionNumbers(
                            offset_dims=(), collapsed_slice_dims=(0,),
                            start_index_map=(0,)),
                        slice_sizes=(1,),
                        mode=lax.GatherScatterMode.PROMISE_IN_BOUNDS)
                    for d in range(D // 16):
                        rsv[e, pl.ds(d * 16, 16)] = (
                            rsv[e, pl.ds(d * 16, 16)] * vv)

            pltpu.async_copy(rsv, acc.at[rv], sem_s, add=True).wait()

        @pl.loop(0, NFULL)
        def _(i):
            process_chunk(base + i * CHUNK, CHUNK,
                          row_v, col_v, dat_v, degg_v, scl_v, rows_v)

        process_chunk(base + NFULL * CHUNK, TAIL,
                      rowt_v, colt_v, datt_v, degt_v, sclt_v, rowst_v)

        plsc.subcore_barrier()

        @pl.loop(0, 3)
        def _(i):
            r0 = s * 624 + i * 208
            pltpu.sync_copy(acc.at[pl.ds(r0, 208)], zbuf)
            pltpu.sync_copy(zbuf, out_hbm.at[c, pl.ds(r0, 208)])

        @pl.when(s == NS - 1)
        def _():
            pltpu.sync_copy(acc.at[pl.ds(9984, 16)], zbuf.at[pl.ds(0, 16)])
            pltpu.sync_copy(zbuf.at[pl.ds(0, 16)],
                            out_hbm.at[c, pl.ds(9984, 16)])

    return k(adj_row, adj_col, adj_data, deg, h)


def _tc1_body(dp_ref, emb_ref, w1t_ref, wa_ref, ba_ref, deg_ref, h_ref):
    deg_ref[...] = dp_ref[0] + dp_ref[1]
    x = emb_ref[...]
    scores = jnp.dot(x, wa_ref[...], preferred_element_type=jnp.float32)
    scores = scores + ba_ref[0, 0]
    m = jnp.max(scores)
    ex = jnp.exp(scores - m)
    attn = ex / jnp.sum(ex)
    xw = jnp.dot(x, w1t_ref[...], preferred_element_type=jnp.float32)
    h_ref[...] = xw * attn


def _tc1(deg_partials, emb, w1t, wa, ba):
    return pl.pallas_call(
        _tc1_body,
        out_shape=(jax.ShapeDtypeStruct((N,), jnp.float32),
                   jax.ShapeDtypeStruct((N, D), jnp.float32)),
    )(deg_partials, emb, w1t, wa, ba)


def _norm_rows(x):
    nrm = jnp.sqrt(jnp.sum(x * x, axis=-1, keepdims=True))
    return x / jnp.maximum(nrm, 1e-12)


def _tc2_body(p_ref, emb_ref, w2t_ref, wa_ref, ba_ref, f1_ref, h_ref):
    x = _norm_rows(p_ref[0] + p_ref[1])
    f1_ref[...] = emb_ref[...] + x
    scores = jnp.dot(x, wa_ref[...], preferred_element_type=jnp.float32)
    scores = scores + ba_ref[0, 0]
    m = jnp.max(scores)
    ex = jnp.exp(scores - m)
    attn = ex / jnp.sum(ex)
    xw = jnp.dot(x, w2t_ref[...], preferred_element_type=jnp.float32)
    h_ref[...] = xw * attn


def _tc2(partials, emb, w2t, wa, ba):
    return pl.pallas_call(
        _tc2_body,
        out_shape=(jax.ShapeDtypeStruct((N, D), jnp.float32),
                   jax.ShapeDtypeStruct((N, D), jnp.float32)),
    )(partials, emb, w2t, wa, ba)


def _tc3_body(p_ref, f1_ref, out_ref):
    x = _norm_rows(p_ref[0] + p_ref[1])
    out_ref[...] = (f1_ref[...] + x) * (1.0 / (L + 1))


def _tc3(partials, f1):
    return pl.pallas_call(
        _tc3_body,
        out_shape=jax.ShapeDtypeStruct((N, D), jnp.float32),
    )(partials, f1)


def kernel(adj_row, adj_col, adj_data, embedding, W_item, W_att, b_att):
    adj_row = adj_row.astype(jnp.int32)
    adj_col = adj_col.astype(jnp.int32)
    ba = b_att.reshape(1, 1)

    deg_partials = _sc_degree(adj_col, adj_data).reshape(NC, N)
    deg, h1 = _tc1(deg_partials, embedding, W_item[0].T, W_att, ba)
    p1 = _sc_spmm(adj_row, adj_col, adj_data, deg, h1)
    f1, h2 = _tc2(p1, embedding, W_item[1].T, W_att, ba)
    p2 = _sc_spmm(adj_row, adj_col, adj_data, deg, h2)
    return _tc3(p2, f1)
